# baseline (speedup 1.0000x reference)
import jax
import jax.numpy as jnp
from jax.experimental import pallas as pl
from jax.experimental.pallas import tpu as pltpu

KNEIGHBORS = 24


def _norm_concat_kernel(gf_ref, sf_ref, stats_ref, alpha_ref, beta_ref, out_ref):
    gf = gf_ref[0]
    sf = sf_ref[0]
    c = gf.shape[-1]
    inv = stats_ref[pl.program_id(0), 0]
    scaled = gf * inv * alpha_ref[...] + beta_ref[...]
    out_ref[0, :, :c] = scaled
    out_ref[0, :, c:] = sf


def kernel(points, points_features, fps_idx, alpha, beta):
    batch, n, _ = points.shape
    fps_num = fps_idx.shape[1]
    k = KNEIGHBORS
    c = points_features.shape[-1]

    sample_points = jnp.take_along_axis(points, fps_idx[:, :, None], axis=1)
    sample_features = jnp.take_along_axis(points_features, fps_idx[:, :, None], axis=1)

    dist1 = jnp.sum(points**2, -1).reshape(batch, 1, n)
    dist2 = jnp.sum(sample_points**2, -1).reshape(batch, fps_num, 1)
    dist3 = jnp.matmul(sample_points, jnp.transpose(points, (0, 2, 1)))
    dist = dist1 + dist2 - 2.0 * dist3
    _, group_idx = jax.lax.top_k(-dist, k)

    bidx = jnp.arange(batch)[:, None, None]
    group_features = points_features[bidx, group_idx, :]
    mean = sample_features[:, :, None, :]
    diff = group_features - mean
    m = fps_num * k * c
    s = jnp.sum(diff.reshape(batch, -1), axis=-1)
    ss = jnp.sum(diff.reshape(batch, -1) ** 2, axis=-1)
    var = (ss - s * s / m) / (m - 1)
    std = jnp.sqrt(var)
    inv = 1.0 / (std + 1e-5)

    diff2 = diff.reshape(batch, fps_num * k, c)
    sf = jnp.broadcast_to(sample_features[:, :, None, :], (batch, fps_num, k, c))
    sf2 = sf.reshape(batch, fps_num * k, c)

    rows = fps_num * k
    blk = 2048
    out = pl.pallas_call(
        _norm_concat_kernel,
        grid=(batch, rows // blk),
        in_specs=[
            pl.BlockSpec((1, blk, c), lambda b, i: (b, i, 0)),
            pl.BlockSpec((1, blk, c), lambda b, i: (b, i, 0)),
            pl.BlockSpec((batch, 1), lambda b, i: (0, 0), memory_space=pltpu.SMEM),
            pl.BlockSpec((1, c), lambda b, i: (0, 0)),
            pl.BlockSpec((1, c), lambda b, i: (0, 0)),
        ],
        out_specs=pl.BlockSpec((1, blk, 2 * c), lambda b, i: (b, i, 0)),
        out_shape=jax.ShapeDtypeStruct((batch, rows, 2 * c), jnp.float32),
    )(diff2, sf2, inv[:, None], alpha.reshape(1, c), beta.reshape(1, c))

    out_features = out.reshape(batch, fps_num, k, 2 * c)
    return (fps_idx, sample_points, out_features)


# speedup vs baseline: 2.8346x; 2.8346x over previous
import functools

import jax
import jax.numpy as jnp
from jax import lax
from jax.experimental import pallas as pl
from jax.experimental.pallas import tpu as pltpu
from jax.experimental.pallas import tpu_sc as plsc

B, N, C = 4, 8192, 384
F = 1024
K = 24
NQ = B * F
BIG = 2**30

_info = plsc.get_sparse_core_info()
NC, NS = _info.num_cores, _info.num_subcores
NW = NC * NS
QPW = NQ // NW


def _lane():
    return lax.broadcasted_iota(jnp.int32, (16,), 0)


def _extract(vec, lane, big):
    return jnp.min(jnp.where(_lane() == lane, vec, big))


def _sq_kernel(pT_ref, o_ref):
    px = pT_ref[0, 0, :][None, :]
    py = pT_ref[0, 1, :][None, :]
    pz = pT_ref[0, 2, :][None, :]
    o_ref[0] = (px * px + py * py) + pz * pz


@functools.partial(jax.jit, static_argnums=1)
def _pallas_sq(pT, W):
    return pl.pallas_call(
        _sq_kernel,
        grid=(B,),
        in_specs=[pl.BlockSpec((1, 3, W), lambda b: (b, 0, 0))],
        out_specs=pl.BlockSpec((1, 1, W), lambda b: (b, 0, 0)),
        out_shape=jax.ShapeDtypeStruct((B, 1, W), jnp.float32),
    )(pT)


def _dist_kernel(sp_ref, pT_ref, d1_ref, d2_ref, o_ref):
    d3 = jnp.dot(sp_ref[0], pT_ref[0], preferred_element_type=jnp.float32)
    o_ref[0] = (d1_ref[0] + d2_ref[0]) - 2.0 * d3


@jax.jit
def _pallas_dist(sp, pT, d1, d2):
    QB, NB = 256, 2048
    return pl.pallas_call(
        _dist_kernel,
        grid=(B, F // QB, N // NB),
        in_specs=[
            pl.BlockSpec((1, QB, 3), lambda b, q, n: (b, q, 0)),
            pl.BlockSpec((1, 3, NB), lambda b, q, n: (b, 0, n)),
            pl.BlockSpec((1, 1, NB), lambda b, q, n: (b, 0, n)),
            pl.BlockSpec((1, QB, 1), lambda b, q, n: (b, q, 0)),
        ],
        out_specs=pl.BlockSpec((1, QB, NB), lambda b, q, n: (b, q, n)),
        out_shape=jax.ShapeDtypeStruct((B, F, N), jnp.float32),
    )(sp, pT, d1, d2)


_mesh = plsc.VectorSubcoreMesh(core_axis_name="c", subcore_axis_name="s")


@functools.partial(
    pl.kernel,
    mesh=_mesh,
    compiler_params=pltpu.CompilerParams(needs_layout_passes=False),
    out_type=[
        jax.ShapeDtypeStruct((NQ, C), jnp.float32),
        jax.ShapeDtypeStruct((NQ * 3,), jnp.float32),
    ],
    scratch_types=[
        pltpu.VMEM((16,), jnp.int32),
        pltpu.VMEM((16, C), jnp.float32),
        pltpu.VMEM((16, 128), jnp.float32),
        pltpu.VMEM((48,), jnp.float32),
        pltpu.SemaphoreType.DMA,
        pltpu.SemaphoreType.DMA,
    ],
)
def _k_sample(feat_hbm, pts_hbm, idx_hbm, sf_out, sp_out, idx_v, feat_v,
              pts_v, sp_v, sem1, sem2):
    w = lax.axis_index("s") * NC + lax.axis_index("c")
    lane = _lane()

    def chunk(i, carry):
        base = w * QPW + i * 16
        pltpu.sync_copy(idx_hbm.at[pl.ds(base, 16)], idx_v)
        cp1 = pltpu.async_copy(feat_hbm.at[idx_v], feat_v, sem1)
        cp2 = pltpu.async_copy(pts_hbm.at[idx_v], pts_v, sem2)
        cp1.wait()
        pltpu.sync_copy(feat_v, sf_out.at[pl.ds(base, 16)])
        cp2.wait()
        for c in range(3):
            col = plsc.load_gather(pts_v, [lane, lane * 0 + c])
            plsc.store_scatter(sp_v, [lane * 3 + c], col)
        pltpu.sync_copy(sp_v, sp_out.at[pl.ds(base * 3, 48)])
        return carry

    lax.fori_loop(0, QPW // 16, chunk, 0)


CAP = N + 16


@functools.partial(
    pl.kernel,
    mesh=_mesh,
    compiler_params=pltpu.CompilerParams(needs_layout_passes=False),
    out_type=jax.ShapeDtypeStruct((NQ, 32), jnp.int32),
    scratch_types=[
        pltpu.VMEM((N,), jnp.float32),
        pltpu.VMEM((CAP,), jnp.float32),
        pltpu.VMEM((CAP,), jnp.int32),
        pltpu.VMEM((32,), jnp.int32),
    ],
)
def _k_select(dist_hbm, idx_out, row_v, cval, cidx, outb):
    w = lax.axis_index("s") * NC + lax.axis_index("c")
    lane = _lane()
    inf = jnp.float32(jnp.inf)
    infv = jnp.full((16,), inf, jnp.float32)
    bigv = jnp.full((16,), BIG, jnp.int32)

    def row_body(r, carry):
        row = w * QPW + r
        pltpu.sync_copy(dist_hbm.at[pl.ds(row * N, N)], row_v)

        def p1(i, mm):
            m1, m2 = mm
            v = row_v[pl.ds(i * 16, 16)]
            nm1 = jnp.minimum(m1, v)
            m2 = jnp.minimum(m2, jnp.maximum(m1, v))
            return (nm1, m2)

        m1, m2 = lax.fori_loop(0, N // 16, p1, (infv, infv))
        s1 = jnp.sort(m1)
        r2 = lax.rev(jnp.sort(m2), (0,))
        hi = jnp.sort(jnp.maximum(s1, r2))
        t = _extract(hi, 7, inf)

        def p2(i, off):
            v = row_v[pl.ds(i * 16, 16)]
            mask = v <= t
            cnt = jnp.max(plsc.all_reduce_population_count(mask))

            @pl.when(cnt > 0)
            def _():
                pos = off + plsc.cumsum(mask.astype(jnp.int32)) - 1
                plsc.store_scatter(cval, [pos], v, mask=mask)
                plsc.store_scatter(cidx, [pos], lane + i * 16, mask=mask)

            return off + cnt

        m = lax.fori_loop(0, N // 16, p2, jnp.int32(0))
        plsc.store_scatter(cval, [m + lane], infv)
        plsc.store_scatter(cidx, [m + lane], bigv)
        nv = (m + 15) // 16

        def extract(j, carry):
            o0, o1 = carry

            def scan(vi, acc):
                av, ai = acc
                v = cval[pl.ds(vi * 16, 16)]
                ix = cidx[pl.ds(vi * 16, 16)]
                lt = (v < av) | ((v == av) & (ix < ai))
                return (jnp.where(lt, v, av), jnp.where(lt, ix, ai))

            av, ai = lax.fori_loop(0, nv, scan, (infv, bigv))
            hv = jnp.min(av)
            hix = jnp.min(jnp.where(av == hv, ai, BIG))

            def remove(vi, c):
                ix = cidx[pl.ds(vi * 16, 16)]
                v = cval[pl.ds(vi * 16, 16)]
                cval[pl.ds(vi * 16, 16)] = jnp.where(ix == hix, inf, v)
                return c

            lax.fori_loop(0, nv, remove, 0)
            o0 = jnp.where((lane == j) & (j < 16), hix, o0)
            o1 = jnp.where(lane == (j - 16), hix, o1)
            return (o0, o1)

        o0, o1 = lax.fori_loop(0, K, extract, (bigv, bigv))
        outb[pl.ds(0, 16)] = o0
        outb[pl.ds(16, 16)] = o1
        pltpu.sync_copy(outb, idx_out.at[row])
        return carry

    lax.fori_loop(0, QPW, row_body, 0)


NCV = C // 16


@functools.partial(
    pl.kernel,
    mesh=_mesh,
    compiler_params=pltpu.CompilerParams(needs_layout_passes=False),
    out_type=jax.ShapeDtypeStruct((NW, 16), jnp.float32),
    scratch_types=[
        pltpu.VMEM((32,), jnp.int32),
        pltpu.VMEM((K, C), jnp.float32),
        pltpu.VMEM((C,), jnp.float32),
        pltpu.VMEM((16,), jnp.float32),
        pltpu.SemaphoreType.DMA,
    ],
)
def _k_stats(feat_hbm, gidx_hbm, sf_hbm, part_out, idx_v, g_v, m_v, acc_v, sem):
    w = lax.axis_index("s") * NC + lax.axis_index("c")
    zero = jnp.zeros((16,), jnp.float32)

    def qbody(q, carry):
        s_tot, ss_tot = carry
        row = w * QPW + q
        pltpu.sync_copy(gidx_hbm.at[row], idx_v)
        idx24 = idx_v.at[pl.ds(0, K)]
        cp = pltpu.async_copy(feat_hbm.at[idx24], g_v, sem)
        pltpu.sync_copy(sf_hbm.at[row], m_v)
        cp.wait()

        def cbody(c, carry2):
            s, ss = carry2
            vm = m_v[pl.ds(c * 16, 16)]

            def rbody(r, carry3):
                s3, ss3 = carry3
                g = g_v[r, pl.ds(c * 16, 16)]
                d = g - vm
                return (s3 + d, ss3 + d * d)

            return lax.fori_loop(0, K, rbody, (s, ss))

        return lax.fori_loop(0, NCV, cbody, (s_tot, ss_tot))

    s_tot, ss_tot = lax.fori_loop(0, QPW, qbody, (zero, zero))
    lane = _lane()
    out = jnp.where(lane == 0, jnp.sum(s_tot), jnp.float32(0.0))
    out = jnp.where(lane == 1, jnp.sum(ss_tot), out)
    acc_v[...] = out
    pltpu.sync_copy(acc_v, part_out.at[w])


@functools.partial(
    pl.kernel,
    mesh=_mesh,
    compiler_params=pltpu.CompilerParams(needs_layout_passes=False),
    out_type=jax.ShapeDtypeStruct((NQ, K, 2 * C), jnp.float32),
    scratch_types=[
        pltpu.VMEM((32,), jnp.int32),
        pltpu.VMEM((K, C), jnp.float32),
        pltpu.VMEM((C,), jnp.float32),
        pltpu.VMEM((C,), jnp.float32),
        pltpu.VMEM((C,), jnp.float32),
        pltpu.VMEM((C,), jnp.float32),
        pltpu.VMEM((16,), jnp.float32),
        pltpu.VMEM((K, 2 * C), jnp.float32),
        pltpu.SemaphoreType.DMA,
    ],
)
def _k_output(feat_hbm, gidx_hbm, sf_hbm, alpha_hbm, beta_hbm, inv_hbm,
              out_hbm, idx_v, g_v, m_v, al_v, be_v, ap_v, inv_v, ob_v, sem):
    w = lax.axis_index("s") * NC + lax.axis_index("c")
    b = w // (NW // B)
    pltpu.sync_copy(alpha_hbm, al_v)
    pltpu.sync_copy(beta_hbm, be_v)
    pltpu.sync_copy(inv_hbm, inv_v)
    binv = _extract(inv_v[...], b, jnp.float32(jnp.inf))

    def prep(c, carry):
        ap_v[pl.ds(c * 16, 16)] = al_v[pl.ds(c * 16, 16)] * binv
        return carry

    lax.fori_loop(0, NCV, prep, 0)

    def qbody(q, carry):
        row = w * QPW + q
        pltpu.sync_copy(gidx_hbm.at[row], idx_v)
        idx24 = idx_v.at[pl.ds(0, K)]
        cp = pltpu.async_copy(feat_hbm.at[idx24], g_v, sem)
        pltpu.sync_copy(sf_hbm.at[row], m_v)
        cp.wait()

        def cbody(c, carry2):
            va = ap_v[pl.ds(c * 16, 16)]
            vm = m_v[pl.ds(c * 16, 16)]
            vb = be_v[pl.ds(c * 16, 16)] - va * vm

            def rbody(r, carry3):
                g = g_v[r, pl.ds(c * 16, 16)]
                ob_v[r, pl.ds(c * 16, 16)] = va * g + vb
                ob_v[r, pl.ds(C + c * 16, 16)] = vm
                return carry3

            lax.fori_loop(0, K, rbody, 0)
            return carry2

        lax.fori_loop(0, NCV, cbody, 0)
        pltpu.sync_copy(ob_v, out_hbm.at[row])
        return carry

    lax.fori_loop(0, QPW, qbody, 0)


def kernel(points, points_features, fps_idx, alpha, beta):
    pT = jnp.transpose(points, (0, 2, 1))
    feat_flat = points_features.reshape(B * N, C)
    pts_pad = jnp.pad(points, ((0, 0), (0, 0), (0, 125))).reshape(B * N, 128)
    fps_flat = (fps_idx + (jnp.arange(B, dtype=jnp.int32) * N)[:, None]).reshape(-1)

    sf_flat, sp_flat = _k_sample(feat_flat, pts_pad, fps_flat)
    sample_points = sp_flat.reshape(B, F, 3)
    sample_features = sf_flat.reshape(B, F, C)

    d1 = _pallas_sq(pT, N)
    spT = jnp.transpose(sample_points, (0, 2, 1))
    d2 = jnp.transpose(_pallas_sq(spT, F), (0, 2, 1))
    dist = _pallas_dist(sample_points, pT, d1, d2)

    gidx = _k_select(dist.reshape(NQ * N))
    gidx_flat = gidx + (jnp.arange(NQ, dtype=jnp.int32) // F * N)[:, None]

    parts = _k_stats(feat_flat, gidx_flat, sf_flat)
    pb = parts.reshape(B, NW // B, 16)
    s = jnp.sum(pb[:, :, 0], axis=1)
    ss = jnp.sum(pb[:, :, 1], axis=1)
    m = F * K * C
    std = jnp.sqrt((ss - s * s / m) / (m - 1))
    inv = 1.0 / (std + 1e-5)
    inv_pad = jnp.pad(inv, (0, 16 - B)).astype(jnp.float32)

    out = _k_output(feat_flat, gidx_flat, sf_flat, alpha.reshape(C),
                    beta.reshape(C), inv_pad)
    out_features = out.reshape(B, F, K, 2 * C)
    return (fps_idx, sample_points, out_features)


# speedup vs baseline: 5.5165x; 1.9461x over previous
import functools

import jax
import jax.numpy as jnp
from jax import lax
from jax.experimental import pallas as pl
from jax.experimental.pallas import tpu as pltpu
from jax.experimental.pallas import tpu_sc as plsc

B, N, C = 4, 8192, 384
F = 1024
K = 24
NQ = B * F
BIG = 2**30

_info = plsc.get_sparse_core_info()
NC, NS = _info.num_cores, _info.num_subcores
NW = NC * NS
QPW = NQ // NW


def _lane():
    return lax.broadcasted_iota(jnp.int32, (16,), 0)


def _extract(vec, lane, big):
    return jnp.min(jnp.where(_lane() == lane, vec, big))


def _sq_kernel(pT_ref, o_ref):
    px = pT_ref[0, 0, :][None, :]
    py = pT_ref[0, 1, :][None, :]
    pz = pT_ref[0, 2, :][None, :]
    o_ref[0] = (px * px + py * py) + pz * pz


@functools.partial(jax.jit, static_argnums=1)
def _pallas_sq(pT, W):
    return pl.pallas_call(
        _sq_kernel,
        grid=(B,),
        in_specs=[pl.BlockSpec((1, 3, W), lambda b: (b, 0, 0))],
        out_specs=pl.BlockSpec((1, 1, W), lambda b: (b, 0, 0)),
        out_shape=jax.ShapeDtypeStruct((B, 1, W), jnp.float32),
    )(pT)


def _dist_kernel(sp_ref, pT_ref, d1_ref, d2_ref, o_ref):
    d3 = jnp.dot(sp_ref[0], pT_ref[0], preferred_element_type=jnp.float32)
    o_ref[0] = (d1_ref[0] + d2_ref[0]) - 2.0 * d3


@jax.jit
def _pallas_dist(sp, pT, d1, d2):
    QB, NB = 256, 2048
    return pl.pallas_call(
        _dist_kernel,
        grid=(B, F // QB, N // NB),
        in_specs=[
            pl.BlockSpec((1, QB, 3), lambda b, q, n: (b, q, 0)),
            pl.BlockSpec((1, 3, NB), lambda b, q, n: (b, 0, n)),
            pl.BlockSpec((1, 1, NB), lambda b, q, n: (b, 0, n)),
            pl.BlockSpec((1, QB, 1), lambda b, q, n: (b, q, 0)),
        ],
        out_specs=pl.BlockSpec((1, QB, NB), lambda b, q, n: (b, q, n)),
        out_shape=jax.ShapeDtypeStruct((B, F, N), jnp.float32),
    )(sp, pT, d1, d2)


_mesh = plsc.VectorSubcoreMesh(core_axis_name="c", subcore_axis_name="s")


@functools.partial(
    pl.kernel,
    mesh=_mesh,
    compiler_params=pltpu.CompilerParams(needs_layout_passes=False),
    out_type=[
        jax.ShapeDtypeStruct((NQ, C), jnp.float32),
        jax.ShapeDtypeStruct((NQ * 3,), jnp.float32),
    ],
    scratch_types=[
        pltpu.VMEM((16,), jnp.int32),
        pltpu.VMEM((16, C), jnp.float32),
        pltpu.VMEM((16, 128), jnp.float32),
        pltpu.VMEM((48,), jnp.float32),
        pltpu.SemaphoreType.DMA,
        pltpu.SemaphoreType.DMA,
    ],
)
def _k_sample(feat_hbm, pts_hbm, idx_hbm, sf_out, sp_out, idx_v, feat_v,
              pts_v, sp_v, sem1, sem2):
    w = lax.axis_index("s") * NC + lax.axis_index("c")
    lane = _lane()

    def chunk(i, carry):
        base = w * QPW + i * 16
        pltpu.sync_copy(idx_hbm.at[pl.ds(base, 16)], idx_v)
        cp1 = pltpu.async_copy(feat_hbm.at[idx_v], feat_v, sem1)
        cp2 = pltpu.async_copy(pts_hbm.at[idx_v], pts_v, sem2)
        cp1.wait()
        pltpu.sync_copy(feat_v, sf_out.at[pl.ds(base, 16)])
        cp2.wait()
        for c in range(3):
            col = plsc.load_gather(pts_v, [lane, lane * 0 + c])
            plsc.store_scatter(sp_v, [lane * 3 + c], col)
        pltpu.sync_copy(sp_v, sp_out.at[pl.ds(base * 3, 48)])
        return carry

    lax.fori_loop(0, QPW // 16, chunk, 0)


CAP = N + 16


@functools.partial(
    pl.kernel,
    mesh=_mesh,
    compiler_params=pltpu.CompilerParams(needs_layout_passes=False),
    out_type=jax.ShapeDtypeStruct((NQ, 32), jnp.int32),
    scratch_types=[
        pltpu.VMEM((N,), jnp.float32),
        pltpu.VMEM((N,), jnp.float32),
        pltpu.VMEM((CAP,), jnp.float32),
        pltpu.VMEM((CAP,), jnp.int32),
        pltpu.VMEM((32,), jnp.int32),
        pltpu.SemaphoreType.DMA,
        pltpu.SemaphoreType.DMA,
    ],
)
def _k_select(dist_hbm, idx_out, rbuf0, rbuf1, cval, cidx, outb, sem0, sem1):
    w = lax.axis_index("s") * NC + lax.axis_index("c")
    lane = _lane()
    inf = jnp.float32(jnp.inf)
    infv = jnp.full((16,), inf, jnp.float32)
    bigv = jnp.full((16,), BIG, jnp.int32)
    U = 8
    last = w * QPW + QPW - 1

    def process(row_v, row):
        def p1(i, mm):
            m1, m2 = mm
            for u in range(U):
                v = row_v[pl.ds((i * U + u) * 16, 16)]
                nm1 = jnp.minimum(m1, v)
                m2 = jnp.minimum(m2, jnp.maximum(m1, v))
                m1 = nm1
            return (m1, m2)

        m1, m2 = lax.fori_loop(0, N // (16 * U), p1, (infv, infv))
        s1 = jnp.sort(m1)
        r2 = lax.rev(jnp.sort(m2), (0,))
        hi = jnp.sort(jnp.maximum(s1, r2))
        t = _extract(hi, 7, inf)

        def p2(i, off):
            vs, masks, cnts = [], [], []
            for u in range(U):
                v = row_v[pl.ds((i * U + u) * 16, 16)]
                mask = v <= t
                vs.append(v)
                masks.append(mask)
                cnts.append(plsc.all_reduce_population_count(mask))
            tot_v = cnts[0]
            for u in range(1, U):
                tot_v = tot_v + cnts[u]
            total = jnp.max(tot_v)

            @pl.when(total > 0)
            def _():
                o = off
                for u in range(U):
                    pos = o + plsc.cumsum(masks[u].astype(jnp.int32)) - 1
                    plsc.store_scatter(cval, [pos], vs[u], mask=masks[u])
                    plsc.store_scatter(cidx, [pos], lane + (i * U + u) * 16,
                                       mask=masks[u])
                    o = o + jnp.max(cnts[u])

            return off + total

        m = lax.fori_loop(0, N // (16 * U), p2, jnp.int32(0))
        plsc.store_scatter(cval, [m + lane], infv)
        plsc.store_scatter(cidx, [m + lane], bigv)
        nv = (m + 15) // 16

        def extract(j, carry):
            o0, o1, prev = carry

            def scan(vi, acc):
                av, ai = acc
                v = cval[pl.ds(vi * 16, 16)]
                ix = cidx[pl.ds(vi * 16, 16)]
                v = jnp.where(ix == prev, inf, v)
                cval[pl.ds(vi * 16, 16)] = v
                lt = (v < av) | ((v == av) & (ix < ai))
                return (jnp.where(lt, v, av), jnp.where(lt, ix, ai))

            av, ai = lax.fori_loop(0, nv, scan, (infv, bigv))
            hv = jnp.min(av)
            hix = jnp.min(jnp.where(av == hv, ai, BIG))
            o0 = jnp.where((lane == j) & (j < 16), hix, o0)
            o1 = jnp.where(lane == (j - 16), hix, o1)
            return (o0, o1, hix)

        o0, o1, _unused = lax.fori_loop(0, K, extract, (bigv, bigv, jnp.int32(-1)))
        outb[pl.ds(0, 16)] = o0
        outb[pl.ds(16, 16)] = o1
        pltpu.sync_copy(outb, idx_out.at[row])

    pltpu.async_copy(dist_hbm.at[pl.ds(w * QPW * N, N)], rbuf0, sem0).wait()

    def pair(p, carry):
        r0 = w * QPW + 2 * p
        nxt1 = jnp.minimum(r0 + 1, last)
        cp1 = pltpu.async_copy(dist_hbm.at[pl.ds(nxt1 * N, N)], rbuf1, sem1)
        process(rbuf0, r0)
        cp1.wait()
        nxt2 = jnp.minimum(r0 + 2, last)
        cp0 = pltpu.async_copy(dist_hbm.at[pl.ds(nxt2 * N, N)], rbuf0, sem0)
        process(rbuf1, r0 + 1)
        cp0.wait()
        return carry

    lax.fori_loop(0, QPW // 2, pair, 0)


NCV = C // 16


@functools.partial(
    pl.kernel,
    mesh=_mesh,
    compiler_params=pltpu.CompilerParams(needs_layout_passes=False),
    out_type=jax.ShapeDtypeStruct((NW, 16), jnp.float32),
    scratch_types=[
        pltpu.VMEM((32,), jnp.int32),
        pltpu.VMEM((K, C), jnp.float32),
        pltpu.VMEM((C,), jnp.float32),
        pltpu.VMEM((16,), jnp.float32),
        pltpu.SemaphoreType.DMA,
    ],
)
def _k_stats(feat_hbm, gidx_hbm, sf_hbm, part_out, idx_v, g_v, m_v, acc_v, sem):
    w = lax.axis_index("s") * NC + lax.axis_index("c")
    zero = jnp.zeros((16,), jnp.float32)

    def qbody(q, carry):
        s_tot, ss_tot = carry
        row = w * QPW + q
        pltpu.sync_copy(gidx_hbm.at[row], idx_v)
        idx24 = idx_v.at[pl.ds(0, K)]
        cp = pltpu.async_copy(feat_hbm.at[idx24], g_v, sem)
        pltpu.sync_copy(sf_hbm.at[row], m_v)
        cp.wait()

        def cbody(c, carry2):
            s, ss = carry2
            vm = m_v[pl.ds(c * 16, 16)]

            def rbody(r, carry3):
                s3, ss3 = carry3
                g = g_v[r, pl.ds(c * 16, 16)]
                d = g - vm
                return (s3 + d, ss3 + d * d)

            return lax.fori_loop(0, K, rbody, (s, ss))

        return lax.fori_loop(0, NCV, cbody, (s_tot, ss_tot))

    s_tot, ss_tot = lax.fori_loop(0, QPW, qbody, (zero, zero))
    lane = _lane()
    out = jnp.where(lane == 0, jnp.sum(s_tot), jnp.float32(0.0))
    out = jnp.where(lane == 1, jnp.sum(ss_tot), out)
    acc_v[...] = out
    pltpu.sync_copy(acc_v, part_out.at[w])


@functools.partial(
    pl.kernel,
    mesh=_mesh,
    compiler_params=pltpu.CompilerParams(needs_layout_passes=False),
    out_type=jax.ShapeDtypeStruct((NQ, K, 2 * C), jnp.float32),
    scratch_types=[
        pltpu.VMEM((32,), jnp.int32),
        pltpu.VMEM((K, C), jnp.float32),
        pltpu.VMEM((C,), jnp.float32),
        pltpu.VMEM((C,), jnp.float32),
        pltpu.VMEM((C,), jnp.float32),
        pltpu.VMEM((C,), jnp.float32),
        pltpu.VMEM((16,), jnp.float32),
        pltpu.VMEM((K, 2 * C), jnp.float32),
        pltpu.SemaphoreType.DMA,
    ],
)
def _k_output(feat_hbm, gidx_hbm, sf_hbm, alpha_hbm, beta_hbm, inv_hbm,
              out_hbm, idx_v, g_v, m_v, al_v, be_v, ap_v, inv_v, ob_v, sem):
    w = lax.axis_index("s") * NC + lax.axis_index("c")
    b = w // (NW // B)
    pltpu.sync_copy(alpha_hbm, al_v)
    pltpu.sync_copy(beta_hbm, be_v)
    pltpu.sync_copy(inv_hbm, inv_v)
    binv = _extract(inv_v[...], b, jnp.float32(jnp.inf))

    def prep(c, carry):
        ap_v[pl.ds(c * 16, 16)] = al_v[pl.ds(c * 16, 16)] * binv
        return carry

    lax.fori_loop(0, NCV, prep, 0)

    def qbody(q, carry):
        row = w * QPW + q
        pltpu.sync_copy(gidx_hbm.at[row], idx_v)
        idx24 = idx_v.at[pl.ds(0, K)]
        cp = pltpu.async_copy(feat_hbm.at[idx24], g_v, sem)
        pltpu.sync_copy(sf_hbm.at[row], m_v)
        cp.wait()

        def cbody(c, carry2):
            va = ap_v[pl.ds(c * 16, 16)]
            vm = m_v[pl.ds(c * 16, 16)]
            vb = be_v[pl.ds(c * 16, 16)] - va * vm

            def rbody(r, carry3):
                g = g_v[r, pl.ds(c * 16, 16)]
                ob_v[r, pl.ds(c * 16, 16)] = va * g + vb
                ob_v[r, pl.ds(C + c * 16, 16)] = vm
                return carry3

            lax.fori_loop(0, K, rbody, 0)
            return carry2

        lax.fori_loop(0, NCV, cbody, 0)
        pltpu.sync_copy(ob_v, out_hbm.at[row])
        return carry

    lax.fori_loop(0, QPW, qbody, 0)


def kernel(points, points_features, fps_idx, alpha, beta):
    pT = jnp.transpose(points, (0, 2, 1))
    feat_flat = points_features.reshape(B * N, C)
    pts_pad = jnp.pad(points, ((0, 0), (0, 0), (0, 125))).reshape(B * N, 128)
    fps_flat = (fps_idx + (jnp.arange(B, dtype=jnp.int32) * N)[:, None]).reshape(-1)

    sf_flat, sp_flat = _k_sample(feat_flat, pts_pad, fps_flat)
    sample_points = sp_flat.reshape(B, F, 3)
    sample_features = sf_flat.reshape(B, F, C)

    d1 = _pallas_sq(pT, N)
    spT = jnp.transpose(sample_points, (0, 2, 1))
    d2 = jnp.transpose(_pallas_sq(spT, F), (0, 2, 1))
    dist = _pallas_dist(sample_points, pT, d1, d2)

    gidx = _k_select(dist.reshape(NQ * N))
    gidx_flat = gidx + (jnp.arange(NQ, dtype=jnp.int32) // F * N)[:, None]

    parts = _k_stats(feat_flat, gidx_flat, sf_flat)
    pb = parts.reshape(B, NW // B, 16)
    s = jnp.sum(pb[:, :, 0], axis=1)
    ss = jnp.sum(pb[:, :, 1], axis=1)
    m = F * K * C
    std = jnp.sqrt((ss - s * s / m) / (m - 1))
    inv = 1.0 / (std + 1e-5)
    inv_pad = jnp.pad(inv, (0, 16 - B)).astype(jnp.float32)

    out = _k_output(feat_flat, gidx_flat, sf_flat, alpha.reshape(C),
                    beta.reshape(C), inv_pad)
    out_features = out.reshape(B, F, K, 2 * C)
    return (fps_idx, sample_points, out_features)


# speedup vs baseline: 6.4542x; 1.1700x over previous
import functools

import jax
import jax.numpy as jnp
from jax import lax
from jax.experimental import pallas as pl
from jax.experimental.pallas import tpu as pltpu
from jax.experimental.pallas import tpu_sc as plsc

B, N, C = 4, 8192, 384
F = 1024
K = 24
NQ = B * F
BIG = 2**30

_info = plsc.get_sparse_core_info()
NC, NS = _info.num_cores, _info.num_subcores
NW = NC * NS
QPW = NQ // NW


def _lane():
    return lax.broadcasted_iota(jnp.int32, (16,), 0)


def _extract(vec, lane, big):
    return jnp.min(jnp.where(_lane() == lane, vec, big))


def _sq_kernel(pT_ref, o_ref):
    px = pT_ref[0, 0, :][None, :]
    py = pT_ref[0, 1, :][None, :]
    pz = pT_ref[0, 2, :][None, :]
    o_ref[0] = (px * px + py * py) + pz * pz


@functools.partial(jax.jit, static_argnums=1)
def _pallas_sq(pT, W):
    return pl.pallas_call(
        _sq_kernel,
        grid=(B,),
        in_specs=[pl.BlockSpec((1, 3, W), lambda b: (b, 0, 0))],
        out_specs=pl.BlockSpec((1, 1, W), lambda b: (b, 0, 0)),
        out_shape=jax.ShapeDtypeStruct((B, 1, W), jnp.float32),
    )(pT)


def _dist_kernel(sp_ref, pT_ref, d1_ref, d2_ref, o_ref):
    d3 = jnp.dot(sp_ref[0], pT_ref[0], preferred_element_type=jnp.float32)
    o_ref[0] = (d1_ref[0] + d2_ref[0]) - 2.0 * d3


@jax.jit
def _pallas_dist(sp, pT, d1, d2):
    QB, NB = 256, 2048
    return pl.pallas_call(
        _dist_kernel,
        grid=(B, F // QB, N // NB),
        in_specs=[
            pl.BlockSpec((1, QB, 3), lambda b, q, n: (b, q, 0)),
            pl.BlockSpec((1, 3, NB), lambda b, q, n: (b, 0, n)),
            pl.BlockSpec((1, 1, NB), lambda b, q, n: (b, 0, n)),
            pl.BlockSpec((1, QB, 1), lambda b, q, n: (b, q, 0)),
        ],
        out_specs=pl.BlockSpec((1, QB, NB), lambda b, q, n: (b, q, n)),
        out_shape=jax.ShapeDtypeStruct((B, F, N), jnp.float32),
    )(sp, pT, d1, d2)


_mesh = plsc.VectorSubcoreMesh(core_axis_name="c", subcore_axis_name="s")


@functools.partial(
    pl.kernel,
    mesh=_mesh,
    compiler_params=pltpu.CompilerParams(needs_layout_passes=False),
    out_type=[
        jax.ShapeDtypeStruct((NQ, C), jnp.float32),
        jax.ShapeDtypeStruct((NQ * 3,), jnp.float32),
    ],
    scratch_types=[
        pltpu.VMEM((16,), jnp.int32),
        pltpu.VMEM((16, C), jnp.float32),
        pltpu.VMEM((16, 128), jnp.float32),
        pltpu.VMEM((48,), jnp.float32),
        pltpu.SemaphoreType.DMA,
        pltpu.SemaphoreType.DMA,
    ],
)
def _k_sample(feat_hbm, pts_hbm, idx_hbm, sf_out, sp_out, idx_v, feat_v,
              pts_v, sp_v, sem1, sem2):
    w = lax.axis_index("s") * NC + lax.axis_index("c")
    lane = _lane()

    def chunk(i, carry):
        base = w * QPW + i * 16
        pltpu.sync_copy(idx_hbm.at[pl.ds(base, 16)], idx_v)
        cp1 = pltpu.async_copy(feat_hbm.at[idx_v], feat_v, sem1)
        cp2 = pltpu.async_copy(pts_hbm.at[idx_v], pts_v, sem2)
        cp1.wait()
        pltpu.sync_copy(feat_v, sf_out.at[pl.ds(base, 16)])
        cp2.wait()
        for c in range(3):
            col = plsc.load_gather(pts_v, [lane, lane * 0 + c])
            plsc.store_scatter(sp_v, [lane * 3 + c], col)
        pltpu.sync_copy(sp_v, sp_out.at[pl.ds(base * 3, 48)])
        return carry

    lax.fori_loop(0, QPW // 16, chunk, 0)


CAP = N + 16


@functools.partial(
    pl.kernel,
    mesh=_mesh,
    compiler_params=pltpu.CompilerParams(needs_layout_passes=False),
    out_type=jax.ShapeDtypeStruct((NQ, 32), jnp.int32),
    scratch_types=[
        pltpu.VMEM((N,), jnp.float32),
        pltpu.VMEM((N,), jnp.float32),
        pltpu.VMEM((CAP,), jnp.float32),
        pltpu.VMEM((CAP,), jnp.int32),
        pltpu.VMEM((32,), jnp.int32),
        pltpu.SemaphoreType.DMA,
        pltpu.SemaphoreType.DMA,
    ],
)
def _k_select(dist_hbm, idx_out, rbuf0, rbuf1, cval, cidx, outb, sem0, sem1):
    w = lax.axis_index("s") * NC + lax.axis_index("c")
    lane = _lane()
    inf = jnp.float32(jnp.inf)
    infv = jnp.full((16,), inf, jnp.float32)
    bigv = jnp.full((16,), BIG, jnp.int32)
    U = 8
    last = w * QPW + QPW - 1

    def process(row_v, row):
        def p1(i, mm):
            m1, m2 = mm
            for u in range(U):
                v = row_v[pl.ds((i * U + u) * 16, 16)]
                nm1 = jnp.minimum(m1, v)
                m2 = jnp.minimum(m2, jnp.maximum(m1, v))
                m1 = nm1
            return (m1, m2)

        m1, m2 = lax.fori_loop(0, N // (16 * U), p1, (infv, infv))
        s1 = jnp.sort(m1)
        r2 = lax.rev(jnp.sort(m2), (0,))
        hi = jnp.sort(jnp.maximum(s1, r2))
        t = _extract(hi, 7, inf)

        def p2(i, off):
            vs, masks, cnts = [], [], []
            for u in range(U):
                v = row_v[pl.ds((i * U + u) * 16, 16)]
                mask = v <= t
                vs.append(v)
                masks.append(mask)
                cnts.append(plsc.all_reduce_population_count(mask))
            tot_v = cnts[0]
            for u in range(1, U):
                tot_v = tot_v + cnts[u]
            total = jnp.max(tot_v)

            @pl.when(total > 0)
            def _():
                o = off
                for u in range(U):
                    pos = o + plsc.cumsum(masks[u].astype(jnp.int32)) - 1
                    plsc.store_scatter(cval, [pos], vs[u], mask=masks[u])
                    plsc.store_scatter(cidx, [pos], lane + (i * U + u) * 16,
                                       mask=masks[u])
                    o = o + jnp.max(cnts[u])

            return off + total

        m = lax.fori_loop(0, N // (16 * U), p2, jnp.int32(0))
        plsc.store_scatter(cval, [m + lane], infv)
        plsc.store_scatter(cidx, [m + lane], bigv)
        nv = (m + 15) // 16

        def extract(j, carry):
            o0, o1, prev = carry

            def scan(vi, acc):
                av, ai = acc
                v = cval[pl.ds(vi * 16, 16)]
                ix = cidx[pl.ds(vi * 16, 16)]
                v = jnp.where(ix == prev, inf, v)
                cval[pl.ds(vi * 16, 16)] = v
                lt = (v < av) | ((v == av) & (ix < ai))
                return (jnp.where(lt, v, av), jnp.where(lt, ix, ai))

            av, ai = lax.fori_loop(0, nv, scan, (infv, bigv))
            hv = jnp.min(av)
            hix = jnp.min(jnp.where(av == hv, ai, BIG))
            o0 = jnp.where((lane == j) & (j < 16), hix, o0)
            o1 = jnp.where(lane == (j - 16), hix, o1)
            return (o0, o1, hix)

        o0, o1, _unused = lax.fori_loop(0, K, extract, (bigv, bigv, jnp.int32(-1)))
        outb[pl.ds(0, 16)] = o0
        outb[pl.ds(16, 16)] = o1
        pltpu.sync_copy(outb, idx_out.at[row])

    pltpu.async_copy(dist_hbm.at[pl.ds(w * QPW * N, N)], rbuf0, sem0).wait()

    def pair(p, carry):
        r0 = w * QPW + 2 * p
        nxt1 = jnp.minimum(r0 + 1, last)
        cp1 = pltpu.async_copy(dist_hbm.at[pl.ds(nxt1 * N, N)], rbuf1, sem1)
        process(rbuf0, r0)
        cp1.wait()
        nxt2 = jnp.minimum(r0 + 2, last)
        cp0 = pltpu.async_copy(dist_hbm.at[pl.ds(nxt2 * N, N)], rbuf0, sem0)
        process(rbuf1, r0 + 1)
        cp0.wait()
        return carry

    lax.fori_loop(0, QPW // 2, pair, 0)


NCV = C // 16


@functools.partial(
    pl.kernel,
    mesh=_mesh,
    compiler_params=pltpu.CompilerParams(needs_layout_passes=False),
    out_type=jax.ShapeDtypeStruct((NW, 16), jnp.float32),
    scratch_types=[
        pltpu.VMEM((32,), jnp.int32),
        pltpu.VMEM((32,), jnp.int32),
        pltpu.VMEM((K, C), jnp.float32),
        pltpu.VMEM((K, C), jnp.float32),
        pltpu.VMEM((C,), jnp.float32),
        pltpu.VMEM((C,), jnp.float32),
        pltpu.VMEM((16,), jnp.float32),
        pltpu.SemaphoreType.DMA,
        pltpu.SemaphoreType.DMA,
        pltpu.SemaphoreType.DMA,
        pltpu.SemaphoreType.DMA,
    ],
)
def _k_stats(feat_hbm, gidx_hbm, sf_hbm, part_out, idx0, idx1, g0, g1, m0,
             m1, acc_v, sg0, sg1, sm0, sm1):
    w = lax.axis_index("s") * NC + lax.axis_index("c")
    zero = jnp.zeros((16,), jnp.float32)
    base = w * QPW
    last = base + QPW - 1

    def compute(g_v, m_v, s_tot, ss_tot):
        def cbody(c, carry2):
            s, ss = carry2
            vm = m_v[pl.ds(c * 16, 16)]

            def rbody(r, carry3):
                s3, ss3 = carry3
                g = g_v[r, pl.ds(c * 16, 16)]
                d = g - vm
                return (s3 + d, ss3 + d * d)

            return lax.fori_loop(0, K, rbody, (s, ss))

        return lax.fori_loop(0, NCV, cbody, (s_tot, ss_tot))

    pltpu.sync_copy(gidx_hbm.at[base], idx0)
    cg = pltpu.async_copy(feat_hbm.at[idx0.at[pl.ds(0, K)]], g0, sg0)
    pltpu.sync_copy(sf_hbm.at[base], m0)
    cg.wait()

    def pair(p, carry):
        s, ss = carry
        q0 = base + 2 * p
        q1 = jnp.minimum(q0 + 1, last)
        q2 = jnp.minimum(q0 + 2, last)
        pltpu.sync_copy(gidx_hbm.at[q1], idx1)
        cg1 = pltpu.async_copy(feat_hbm.at[idx1.at[pl.ds(0, K)]], g1, sg1)
        cm1 = pltpu.async_copy(sf_hbm.at[q1], m1, sm1)
        s, ss = compute(g0, m0, s, ss)
        cg1.wait()
        cm1.wait()
        pltpu.sync_copy(gidx_hbm.at[q2], idx0)
        cg0 = pltpu.async_copy(feat_hbm.at[idx0.at[pl.ds(0, K)]], g0, sg0)
        cm0 = pltpu.async_copy(sf_hbm.at[q2], m0, sm0)
        s, ss = compute(g1, m1, s, ss)
        cg0.wait()
        cm0.wait()
        return (s, ss)

    s_tot, ss_tot = lax.fori_loop(0, QPW // 2, pair, (zero, zero))
    lane = _lane()
    out = jnp.where(lane == 0, jnp.sum(s_tot), jnp.float32(0.0))
    out = jnp.where(lane == 1, jnp.sum(ss_tot), out)
    acc_v[...] = out
    pltpu.sync_copy(acc_v, part_out.at[w])


@functools.partial(
    pl.kernel,
    mesh=_mesh,
    compiler_params=pltpu.CompilerParams(needs_layout_passes=False),
    out_type=jax.ShapeDtypeStruct((NQ, K, 2 * C), jnp.float32),
    scratch_types=[
        pltpu.VMEM((32,), jnp.int32),
        pltpu.VMEM((32,), jnp.int32),
        pltpu.VMEM((K, C), jnp.float32),
        pltpu.VMEM((K, C), jnp.float32),
        pltpu.VMEM((C,), jnp.float32),
        pltpu.VMEM((C,), jnp.float32),
        pltpu.VMEM((C,), jnp.float32),
        pltpu.VMEM((C,), jnp.float32),
        pltpu.VMEM((C,), jnp.float32),
        pltpu.VMEM((16,), jnp.float32),
        pltpu.VMEM((K, 2 * C), jnp.float32),
        pltpu.VMEM((K, 2 * C), jnp.float32),
        pltpu.SemaphoreType.DMA,
        pltpu.SemaphoreType.DMA,
        pltpu.SemaphoreType.DMA,
        pltpu.SemaphoreType.DMA,
        pltpu.SemaphoreType.DMA,
        pltpu.SemaphoreType.DMA,
    ],
)
def _k_output(feat_hbm, gidx_hbm, sf_hbm, alpha_hbm, beta_hbm, inv_hbm,
              out_hbm, idx0, idx1, g0, g1, m0, m1, al_v, be_v, ap_v, inv_v,
              ob0, ob1, sg0, sg1, sm0, sm1, so0, so1):
    w = lax.axis_index("s") * NC + lax.axis_index("c")
    b = w // (NW // B)
    base = w * QPW
    last = base + QPW - 1
    pltpu.sync_copy(alpha_hbm, al_v)
    pltpu.sync_copy(beta_hbm, be_v)
    pltpu.sync_copy(inv_hbm, inv_v)
    binv = _extract(inv_v[...], b, jnp.float32(jnp.inf))

    def prep(c, carry):
        ap_v[pl.ds(c * 16, 16)] = al_v[pl.ds(c * 16, 16)] * binv
        return carry

    lax.fori_loop(0, NCV, prep, 0)

    def compute(g_v, m_v, ob_v):
        def cbody(c, carry2):
            va = ap_v[pl.ds(c * 16, 16)]
            vm = m_v[pl.ds(c * 16, 16)]
            vb = be_v[pl.ds(c * 16, 16)] - va * vm

            def rbody(r, carry3):
                g = g_v[r, pl.ds(c * 16, 16)]
                ob_v[r, pl.ds(c * 16, 16)] = va * g + vb
                ob_v[r, pl.ds(C + c * 16, 16)] = vm
                return carry3

            lax.fori_loop(0, K, rbody, 0)
            return carry2

        lax.fori_loop(0, NCV, cbody, 0)

    pltpu.sync_copy(gidx_hbm.at[base], idx0)
    cg = pltpu.async_copy(feat_hbm.at[idx0.at[pl.ds(0, K)]], g0, sg0)
    pltpu.sync_copy(sf_hbm.at[base], m0)
    cg.wait()

    def pair(p, carry):
        q0 = base + 2 * p
        q1 = jnp.minimum(q0 + 1, last)
        q2 = jnp.minimum(q0 + 2, last)
        pltpu.sync_copy(gidx_hbm.at[q1], idx1)
        cg1 = pltpu.async_copy(feat_hbm.at[idx1.at[pl.ds(0, K)]], g1, sg1)
        cm1 = pltpu.async_copy(sf_hbm.at[q1], m1, sm1)
        compute(g0, m0, ob0)
        co0 = pltpu.async_copy(ob0, out_hbm.at[q0], so0)
        cg1.wait()
        cm1.wait()
        pltpu.sync_copy(gidx_hbm.at[q2], idx0)
        cg0 = pltpu.async_copy(feat_hbm.at[idx0.at[pl.ds(0, K)]], g0, sg0)
        cm0 = pltpu.async_copy(sf_hbm.at[q2], m0, sm0)
        compute(g1, m1, ob1)
        co1 = pltpu.async_copy(ob1, out_hbm.at[q0 + 1], so1)
        co0.wait()
        cg0.wait()
        cm0.wait()
        co1.wait()
        return carry

    lax.fori_loop(0, QPW // 2, pair, 0)


def kernel(points, points_features, fps_idx, alpha, beta):
    pT = jnp.transpose(points, (0, 2, 1))
    feat_flat = points_features.reshape(B * N, C)
    pts_pad = jnp.pad(points, ((0, 0), (0, 0), (0, 125))).reshape(B * N, 128)
    fps_flat = (fps_idx + (jnp.arange(B, dtype=jnp.int32) * N)[:, None]).reshape(-1)

    sf_flat, sp_flat = _k_sample(feat_flat, pts_pad, fps_flat)
    sample_points = sp_flat.reshape(B, F, 3)
    sample_features = sf_flat.reshape(B, F, C)

    d1 = _pallas_sq(pT, N)
    spT = jnp.transpose(sample_points, (0, 2, 1))
    d2 = jnp.transpose(_pallas_sq(spT, F), (0, 2, 1))
    dist = _pallas_dist(sample_points, pT, d1, d2)

    gidx = _k_select(dist.reshape(NQ * N))
    gidx_flat = gidx + (jnp.arange(NQ, dtype=jnp.int32) // F * N)[:, None]

    parts = _k_stats(feat_flat, gidx_flat, sf_flat)
    pb = parts.reshape(B, NW // B, 16)
    s = jnp.sum(pb[:, :, 0], axis=1)
    ss = jnp.sum(pb[:, :, 1], axis=1)
    m = F * K * C
    std = jnp.sqrt((ss - s * s / m) / (m - 1))
    inv = 1.0 / (std + 1e-5)
    inv_pad = jnp.pad(inv, (0, 16 - B)).astype(jnp.float32)

    out = _k_output(feat_flat, gidx_flat, sf_flat, alpha.reshape(C),
                    beta.reshape(C), inv_pad)
    out_features = out.reshape(B, F, K, 2 * C)
    return (fps_idx, sample_points, out_features)
